# baseline (speedup 1.0000x reference)
import jax
import jax.numpy as jnp
from jax.experimental import pallas as pl
from jax.experimental.pallas import tpu as pltpu

B, TOTAL, D_Z, D_PAIR, D_LM, M, H = 16, 32768, 128, 128, 768, 16, 128

CHUNK = 4096
N_P1 = TOTAL // CHUNK
NEG = -1e30

BLK = 512
N_P2 = (M * D_LM) // BLK
G = 4
L = N_P2 // G


def _fused_kernel(st_ref, en_ref, zs_ref, wv_ref, wu_ref, w1_ref, b1_ref,
                  b2_ref, zn_ref, ep_ref, w2a_ref, w2b_ref, w2c_ref, w2d_ref,
                  out_ref, m_ref, l_ref, acc_ref, h_ref, weff_ref):
    i = pl.program_id(0)

    @pl.when(i == 0)
    def _init():
        m_ref[...] = jnp.full_like(m_ref, NEG)
        l_ref[...] = jnp.zeros_like(l_ref)
        acc_ref[...] = jnp.zeros_like(acc_ref)
        q = jax.lax.dot_general(zs_ref[...], wv_ref[...],
                                (((1,), (1,)), ((), ())),
                                preferred_element_type=jnp.float32)
        weff_ref[...] = jax.lax.dot_general(
            q, wu_ref[...], (((1,), (0,)), ((), ())),
            preferred_element_type=jnp.float32) * (H ** -0.5)

    @pl.when(i < N_P1)
    def _phase1():
        s = jax.lax.dot_general(weff_ref[...], zn_ref[...],
                                (((1,), (1,)), ((), ())),
                                preferred_element_type=jnp.float32)
        row = i * CHUNK + jax.lax.broadcasted_iota(jnp.int32, (B, CHUNK), 1)
        mask = (row >= st_ref[...]) & (row < en_ref[...])
        s = jnp.where(mask, s, NEG)

        m_prev = m_ref[...]
        m_new = jnp.maximum(m_prev, jnp.max(s, axis=1, keepdims=True))
        p = jnp.exp(s - m_new)
        corr = jnp.exp(m_prev - m_new)
        l_ref[...] = l_ref[...] * corr + jnp.sum(p, axis=1, keepdims=True)
        acc_ref[...] = acc_ref[...] * corr + jax.lax.dot_general(
            p, ep_ref[...], (((1,), (0,)), ((), ())),
            preferred_element_type=jnp.float32)
        m_ref[...] = m_new

        @pl.when(i == N_P1 - 1)
        def _finalize():
            nonempty = en_ref[...] > st_ref[...]
            evx = jnp.where(nonempty, acc_ref[...] / l_ref[...], 0.0)
            h = jax.lax.dot_general(evx, w1_ref[...], (((1,), (1,)), ((), ())),
                                    preferred_element_type=jnp.float32) + b1_ref[...]
            h_ref[...] = 0.5 * h * (1.0 + jax.lax.erf(h * (2.0 ** -0.5)))

    @pl.when(i >= N_P1)
    def _phase2():
        g = (i - N_P1) // L
        for gg, w2_ref in enumerate((w2a_ref, w2b_ref, w2c_ref, w2d_ref)):
            @pl.when(g == gg)
            def _emit(w2_ref=w2_ref):
                out_ref[...] = jax.lax.dot_general(
                    h_ref[...], w2_ref[...], (((1,), (1,)), ((), ())),
                    preferred_element_type=jnp.float32) + b2_ref[...]


def _w2_map(g):
    def _map(i):
        local = jnp.clip(i - N_P1 - g * L, 0, L - 1)
        return (g * L + local, 0)
    return _map


def kernel(Z_self, Z_neigh_flat, E_pair_flat, ptr, Wv, Wu, W1, b1, W2, b2):
    st = ptr[:B].reshape(B, 1)
    en = ptr[1:].reshape(B, 1)

    out = pl.pallas_call(
        _fused_kernel,
        grid=(N_P1 + N_P2,),
        in_specs=[
            pl.BlockSpec((B, 1), lambda i: (0, 0)),
            pl.BlockSpec((B, 1), lambda i: (0, 0)),
            pl.BlockSpec((B, D_Z), lambda i: (0, 0)),
            pl.BlockSpec((H, D_Z), lambda i: (0, 0)),
            pl.BlockSpec((H, D_Z), lambda i: (0, 0)),
            pl.BlockSpec((4 * D_LM, D_PAIR), lambda i: (0, 0)),
            pl.BlockSpec((1, 4 * D_LM), lambda i: (0, 0)),
            pl.BlockSpec((1, BLK), lambda i: (0, jnp.maximum(i - N_P1, 0))),
            pl.BlockSpec((CHUNK, D_Z), lambda i: (jnp.minimum(i, N_P1 - 1), 0)),
            pl.BlockSpec((CHUNK, D_PAIR), lambda i: (jnp.minimum(i, N_P1 - 1), 0)),
            pl.BlockSpec((BLK, 4 * D_LM), _w2_map(0)),
            pl.BlockSpec((BLK, 4 * D_LM), _w2_map(1)),
            pl.BlockSpec((BLK, 4 * D_LM), _w2_map(2)),
            pl.BlockSpec((BLK, 4 * D_LM), _w2_map(3)),
        ],
        out_specs=pl.BlockSpec((B, BLK), lambda i: (0, jnp.maximum(i - N_P1, 0))),
        out_shape=jax.ShapeDtypeStruct((B, M * D_LM), jnp.float32),
        scratch_shapes=[
            pltpu.VMEM((B, 1), jnp.float32),
            pltpu.VMEM((B, 1), jnp.float32),
            pltpu.VMEM((B, D_PAIR), jnp.float32),
            pltpu.VMEM((B, 4 * D_LM), jnp.float32),
            pltpu.VMEM((B, D_Z), jnp.float32),
        ],
        compiler_params=pltpu.CompilerParams(vmem_limit_bytes=112 * 1024 * 1024),
    )(st, en, Z_self, Wv, Wu, W1, b1.reshape(1, -1), b2.reshape(1, -1),
      Z_neigh_flat, E_pair_flat, W2, W2, W2, W2)

    return out.reshape(B, M, D_LM)


# speedup vs baseline: 1.0968x; 1.0968x over previous
import jax
import jax.numpy as jnp
from jax.experimental import pallas as pl
from jax.experimental.pallas import tpu as pltpu

B, TOTAL, D_Z, D_PAIR, D_LM, M, H = 16, 32768, 128, 128, 768, 16, 128

CHUNK = 4096
N_P1 = TOTAL // CHUNK
NEG = -1e30

BLK = 768
N_P2 = (M * D_LM) // BLK
R = 4


def _w2_copy(w2_ref, ring_ref, sem_ref, blk, slot):
    return pltpu.make_async_copy(
        w2_ref.at[pl.ds(blk * BLK, BLK), :], ring_ref.at[slot], sem_ref.at[slot])


def _fused_kernel(st_ref, en_ref, zs_ref, wv_ref, wu_ref, w1_ref, b1_ref,
                  b2_ref, zn_ref, ep_ref, w2_ref, out_ref,
                  m_ref, l_ref, acc_ref, h_ref, weff_ref, ring_ref, sem_ref):
    i = pl.program_id(0)

    @pl.when(i == 0)
    def _init():
        m_ref[...] = jnp.full_like(m_ref, NEG)
        l_ref[...] = jnp.zeros_like(l_ref)
        acc_ref[...] = jnp.zeros_like(acc_ref)
        q = jax.lax.dot_general(zs_ref[...], wv_ref[...],
                                (((1,), (1,)), ((), ())),
                                preferred_element_type=jnp.float32)
        weff_ref[...] = jax.lax.dot_general(
            q, wu_ref[...], (((1,), (0,)), ((), ())),
            preferred_element_type=jnp.float32) * (H ** -0.5)

    @pl.when(i < R)
    def _prefetch():
        _w2_copy(w2_ref, ring_ref, sem_ref, i, i).start()

    @pl.when(i < N_P1)
    def _phase1():
        s = jax.lax.dot_general(weff_ref[...], zn_ref[...],
                                (((1,), (1,)), ((), ())),
                                preferred_element_type=jnp.float32)
        row = i * CHUNK + jax.lax.broadcasted_iota(jnp.int32, (B, CHUNK), 1)
        mask = (row >= st_ref[...]) & (row < en_ref[...])
        s = jnp.where(mask, s, NEG)

        m_prev = m_ref[...]
        m_new = jnp.maximum(m_prev, jnp.max(s, axis=1, keepdims=True))
        p = jnp.exp(s - m_new)
        corr = jnp.exp(m_prev - m_new)
        l_ref[...] = l_ref[...] * corr + jnp.sum(p, axis=1, keepdims=True)
        acc_ref[...] = acc_ref[...] * corr + jax.lax.dot_general(
            p, ep_ref[...], (((1,), (0,)), ((), ())),
            preferred_element_type=jnp.float32)
        m_ref[...] = m_new

        @pl.when(i == N_P1 - 1)
        def _finalize():
            nonempty = en_ref[...] > st_ref[...]
            evx = jnp.where(nonempty, acc_ref[...] / l_ref[...], 0.0)
            h = jax.lax.dot_general(evx, w1_ref[...], (((1,), (1,)), ((), ())),
                                    preferred_element_type=jnp.float32) + b1_ref[...]
            h_ref[...] = 0.5 * h * (1.0 + jax.lax.erf(h * (2.0 ** -0.5)))

    @pl.when(i >= N_P1)
    def _phase2():
        j = i - N_P1
        slot = jax.lax.rem(j, R)
        _w2_copy(w2_ref, ring_ref, sem_ref, j, slot).wait()
        out_ref[...] = jax.lax.dot_general(
            h_ref[...], ring_ref[slot], (((1,), (1,)), ((), ())),
            preferred_element_type=jnp.float32) + b2_ref[...]

        @pl.when(j + R < N_P2)
        def _refill():
            _w2_copy(w2_ref, ring_ref, sem_ref, j + R, slot).start()


def kernel(Z_self, Z_neigh_flat, E_pair_flat, ptr, Wv, Wu, W1, b1, W2, b2):
    st = ptr[:B].reshape(B, 1)
    en = ptr[1:].reshape(B, 1)

    out = pl.pallas_call(
        _fused_kernel,
        grid=(N_P1 + N_P2,),
        in_specs=[
            pl.BlockSpec((B, 1), lambda i: (0, 0)),
            pl.BlockSpec((B, 1), lambda i: (0, 0)),
            pl.BlockSpec((B, D_Z), lambda i: (0, 0)),
            pl.BlockSpec((H, D_Z), lambda i: (0, 0)),
            pl.BlockSpec((H, D_Z), lambda i: (0, 0)),
            pl.BlockSpec((4 * D_LM, D_PAIR), lambda i: (0, 0)),
            pl.BlockSpec((1, 4 * D_LM), lambda i: (0, 0)),
            pl.BlockSpec((1, BLK), lambda i: (0, jnp.maximum(i - N_P1, 0))),
            pl.BlockSpec((CHUNK, D_Z), lambda i: (jnp.minimum(i, N_P1 - 1), 0)),
            pl.BlockSpec((CHUNK, D_PAIR), lambda i: (jnp.minimum(i, N_P1 - 1), 0)),
            pl.BlockSpec(memory_space=pl.ANY),
        ],
        out_specs=pl.BlockSpec((B, BLK), lambda i: (0, jnp.maximum(i - N_P1, 0))),
        out_shape=jax.ShapeDtypeStruct((B, M * D_LM), jnp.float32),
        scratch_shapes=[
            pltpu.VMEM((B, 1), jnp.float32),
            pltpu.VMEM((B, 1), jnp.float32),
            pltpu.VMEM((B, D_PAIR), jnp.float32),
            pltpu.VMEM((B, 4 * D_LM), jnp.float32),
            pltpu.VMEM((B, D_Z), jnp.float32),
            pltpu.VMEM((R, BLK, 4 * D_LM), jnp.float32),
            pltpu.SemaphoreType.DMA((R,)),
        ],
        compiler_params=pltpu.CompilerParams(vmem_limit_bytes=64 * 1024 * 1024),
    )(st, en, Z_self, Wv, Wu, W1, b1.reshape(1, -1), b2.reshape(1, -1),
      Z_neigh_flat, E_pair_flat, W2)

    return out.reshape(B, M, D_LM)


# speedup vs baseline: 1.1987x; 1.0929x over previous
import jax
import jax.numpy as jnp
from jax.experimental import pallas as pl
from jax.experimental.pallas import tpu as pltpu

B, TOTAL, D_Z, D_PAIR, D_LM, M, H = 16, 32768, 128, 128, 768, 16, 128

CHUNK = 8192
N_P1 = TOTAL // CHUNK
NEG = -1e30

BLK = 512
N_P2 = (M * D_LM) // BLK
R = 4


def _w2_copy(w2_ref, ring_ref, sem_ref, blk, slot):
    return pltpu.make_async_copy(
        w2_ref.at[pl.ds(blk * BLK, BLK), :], ring_ref.at[slot], sem_ref.at[slot])


def _fused_kernel(st_ref, en_ref, zs_ref, wv_ref, wu_ref, w1_ref, b1_ref,
                  b2_ref, zn_ref, ep_ref, w2_ref, out_ref,
                  m_ref, l_ref, acc_ref, h_ref, weff_ref, ring_ref, sem_ref):
    i = pl.program_id(0)

    @pl.when(i == 0)
    def _init():
        m_ref[...] = jnp.full_like(m_ref, NEG)
        l_ref[...] = jnp.zeros_like(l_ref)
        acc_ref[...] = jnp.zeros_like(acc_ref)
        q = jax.lax.dot_general(zs_ref[...], wv_ref[...],
                                (((1,), (1,)), ((), ())),
                                preferred_element_type=jnp.float32)
        weff_ref[...] = jax.lax.dot_general(
            q, wu_ref[...], (((1,), (0,)), ((), ())),
            preferred_element_type=jnp.float32) * (H ** -0.5)

    @pl.when(i < R)
    def _prefetch():
        _w2_copy(w2_ref, ring_ref, sem_ref, i, i).start()

    @pl.when(i < N_P1)
    def _phase1():
        s = jax.lax.dot_general(weff_ref[...], zn_ref[...],
                                (((1,), (1,)), ((), ())),
                                preferred_element_type=jnp.float32)
        row = i * CHUNK + jax.lax.broadcasted_iota(jnp.int32, (B, CHUNK), 1)
        mask = (row >= st_ref[...]) & (row < en_ref[...])
        s = jnp.where(mask, s, NEG)

        m_prev = m_ref[...]
        m_new = jnp.maximum(m_prev, jnp.max(s, axis=1, keepdims=True))
        p = jnp.exp(s - m_new)
        corr = jnp.exp(m_prev - m_new)
        l_ref[...] = l_ref[...] * corr + jnp.sum(p, axis=1, keepdims=True)
        acc_ref[...] = acc_ref[...] * corr + jax.lax.dot_general(
            p, ep_ref[...], (((1,), (0,)), ((), ())),
            preferred_element_type=jnp.float32)
        m_ref[...] = m_new

        @pl.when(i == N_P1 - 1)
        def _finalize():
            nonempty = en_ref[...] > st_ref[...]
            evx = jnp.where(nonempty, acc_ref[...] / l_ref[...], 0.0)
            h = jax.lax.dot_general(evx, w1_ref[...], (((1,), (1,)), ((), ())),
                                    preferred_element_type=jnp.float32) + b1_ref[...]
            h_ref[...] = 0.5 * h * (1.0 + jax.lax.erf(h * (2.0 ** -0.5)))

    @pl.when(i >= N_P1)
    def _phase2():
        j = i - N_P1
        slot = jax.lax.rem(j, R)
        _w2_copy(w2_ref, ring_ref, sem_ref, j, slot).wait()
        out_ref[...] = jax.lax.dot_general(
            h_ref[...], ring_ref[slot], (((1,), (1,)), ((), ())),
            preferred_element_type=jnp.float32) + b2_ref[...]

        @pl.when(j + R < N_P2)
        def _refill():
            _w2_copy(w2_ref, ring_ref, sem_ref, j + R, slot).start()


def kernel(Z_self, Z_neigh_flat, E_pair_flat, ptr, Wv, Wu, W1, b1, W2, b2):
    st = ptr[:B].reshape(B, 1)
    en = ptr[1:].reshape(B, 1)

    out = pl.pallas_call(
        _fused_kernel,
        grid=(N_P1 + N_P2,),
        in_specs=[
            pl.BlockSpec((B, 1), lambda i: (0, 0)),
            pl.BlockSpec((B, 1), lambda i: (0, 0)),
            pl.BlockSpec((B, D_Z), lambda i: (0, 0)),
            pl.BlockSpec((H, D_Z), lambda i: (0, 0)),
            pl.BlockSpec((H, D_Z), lambda i: (0, 0)),
            pl.BlockSpec((4 * D_LM, D_PAIR), lambda i: (0, 0)),
            pl.BlockSpec((1, 4 * D_LM), lambda i: (0, 0)),
            pl.BlockSpec((1, BLK), lambda i: (0, jnp.maximum(i - N_P1, 0))),
            pl.BlockSpec((CHUNK, D_Z), lambda i: (jnp.minimum(i, N_P1 - 1), 0)),
            pl.BlockSpec((CHUNK, D_PAIR), lambda i: (jnp.minimum(i, N_P1 - 1), 0)),
            pl.BlockSpec(memory_space=pl.ANY),
        ],
        out_specs=pl.BlockSpec((B, BLK), lambda i: (0, jnp.maximum(i - N_P1, 0))),
        out_shape=jax.ShapeDtypeStruct((B, M * D_LM), jnp.float32),
        scratch_shapes=[
            pltpu.VMEM((B, 1), jnp.float32),
            pltpu.VMEM((B, 1), jnp.float32),
            pltpu.VMEM((B, D_PAIR), jnp.float32),
            pltpu.VMEM((B, 4 * D_LM), jnp.float32),
            pltpu.VMEM((B, D_Z), jnp.float32),
            pltpu.VMEM((R, BLK, 4 * D_LM), jnp.float32),
            pltpu.SemaphoreType.DMA((R,)),
        ],
        compiler_params=pltpu.CompilerParams(vmem_limit_bytes=64 * 1024 * 1024),
    )(st, en, Z_self, Wv, Wu, W1, b1.reshape(1, -1), b2.reshape(1, -1),
      Z_neigh_flat, E_pair_flat, W2)

    return out.reshape(B, M, D_LM)
